# baseline (speedup 1.0000x reference)
import functools

import jax
import jax.numpy as jnp
from jax import lax
from jax.experimental import pallas as pl
from jax.experimental.pallas import tpu as pltpu
from jax.experimental.pallas import tpu_sc as plsc

S = 2048
H = 768
I = 3072
E = 8
TT = 128
NT = 23
NIT = 6
IT = I // NIT
PAD = NT * TT


def _routing_body(x_ref, rw_ref, rb_ref, slot_ref, meta_ref, xbf_ref):
    x = x_ref[...]
    xbf_ref[...] = x.astype(jnp.bfloat16)
    logits = jnp.dot(x, rw_ref[...], preferred_element_type=jnp.float32) + rb_ref[...]
    col = jax.lax.broadcasted_iota(jnp.int32, (S, E), 1)
    m1 = jnp.max(logits, axis=1, keepdims=True)
    i1 = jnp.min(jnp.where(logits == m1, col, E), axis=1, keepdims=True)
    l2 = jnp.where(col == i1, -jnp.inf, logits)
    m2 = jnp.max(l2, axis=1, keepdims=True)
    i2 = jnp.min(jnp.where(l2 == m2, col, E), axis=1, keepdims=True)
    estar = jnp.maximum(i1, i2)

    oh = (estar == col)
    oh_f = oh.astype(jnp.float32)
    oh_b = oh.astype(jnp.bfloat16)
    r_i = jax.lax.broadcasted_iota(jnp.int32, (S, S), 0)
    c_i = jax.lax.broadcasted_iota(jnp.int32, (S, S), 1)
    lt = (c_i < r_i).astype(jnp.bfloat16)
    rank = jnp.dot(lt, oh_b, preferred_element_type=jnp.float32)

    counts = jnp.sum(oh_f, axis=0, keepdims=True).astype(jnp.int32)
    nt = (counts + (TT - 1)) >> 7
    nt_f = nt.astype(jnp.float32)
    a_i = jax.lax.broadcasted_iota(jnp.int32, (E, E), 0)
    b_i = jax.lax.broadcasted_iota(jnp.int32, (E, E), 1)
    m8 = (a_i < b_i).astype(jnp.float32)
    excl_f = jnp.dot(nt_f, m8, preferred_element_type=jnp.float32)
    incl_i = (excl_f + nt_f).astype(jnp.int32)
    start_rows = excl_f * float(TT)

    slot_f = jnp.sum(oh_f * (rank + start_rows), axis=1, keepdims=True)
    slot_ref[...] = slot_f.astype(jnp.int32)

    num_used = jnp.sum(nt, axis=1, keepdims=True)
    col8 = jax.lax.broadcasted_iota(jnp.int32, (1, E), 1)
    maxe = jnp.max(jnp.where(counts > 0, col8, 0), axis=1, keepdims=True)
    jv = jax.lax.broadcasted_iota(jnp.int32, (TT, E), 0)
    raw = jnp.sum((jv >= incl_i).astype(jnp.int32), axis=1, keepdims=True)
    eid = jnp.where(raw == E, maxe, raw)
    jv1 = jax.lax.broadcasted_iota(jnp.int32, (TT, 1), 0)
    meta_ref[...] = jnp.where(jv1 == NT, num_used, eid)


def _ffn_body(eid_ref, nu_ref, slotrow_ref, x_ref, w1_ref, b1_ref, w2_ref,
              b2_ref, out_ref):
    j = pl.program_id(0)

    @pl.when(j < nu_ref[0])
    def _():
        riota = jax.lax.broadcasted_iota(jnp.int32, (TT, S), 0) + j * TT
        oh = (slotrow_ref[...] == riota).astype(jnp.bfloat16)
        xg = jnp.dot(oh, x_ref[...],
                     preferred_element_type=jnp.float32).astype(jnp.bfloat16)
        h = jnp.dot(xg, w1_ref[0].astype(jnp.bfloat16),
                    preferred_element_type=jnp.float32)
        h = h + b1_ref[0]
        h = 0.5 * h * (1.0 + jax.lax.erf(h * 0.7071067811865476))
        out_ref[...] = jnp.dot(h.astype(jnp.bfloat16), w2_ref[0].astype(jnp.bfloat16),
                               preferred_element_type=jnp.float32) + b2_ref[0]


NWORK = 32
RPW = S // NWORK


def _sc_unperm_body(buf_hbm, slot_hbm, out_hbm, idx_v, rows_v, sem):
    wid = lax.axis_index("s") * 2 + lax.axis_index("c")
    base = wid * RPW
    pltpu.sync_copy(slot_hbm.at[pl.ds(base, RPW)], idx_v)
    pltpu.async_copy(buf_hbm.at[idx_v], rows_v, sem).wait()
    pltpu.sync_copy(rows_v, out_hbm.at[pl.ds(base, RPW)])


_sc_unperm = functools.partial(
    pl.kernel,
    mesh=plsc.VectorSubcoreMesh(core_axis_name="c", subcore_axis_name="s"),
    out_type=jax.ShapeDtypeStruct((S, H), jnp.float32),
    scratch_types=[
        pltpu.VMEM((RPW,), jnp.int32),
        pltpu.VMEM((RPW, H), jnp.float32),
        pltpu.SemaphoreType.DMA,
    ],
)(_sc_unperm_body)


def kernel(hidden_states, router_w, router_b, w1, b1, w2, b2):
    x = hidden_states.reshape(S, H)
    rb = router_b.reshape(1, E)

    slot, meta, xbf = pl.pallas_call(
        _routing_body,
        out_shape=[
            jax.ShapeDtypeStruct((S, 1), jnp.int32),
            jax.ShapeDtypeStruct((TT, 1), jnp.int32),
            jax.ShapeDtypeStruct((S, H), jnp.bfloat16),
        ],
    )(x, router_w, rb)

    tile_eid = meta[:NT, 0]
    num_used = meta[NT, 0].reshape(1)
    slot_row = slot.reshape(1, S)

    buf = pl.pallas_call(
        _ffn_body,
        grid_spec=pltpu.PrefetchScalarGridSpec(
            num_scalar_prefetch=2,
            grid=(NT,),
            in_specs=[
                pl.BlockSpec((1, S), lambda j, eid, nu: (0, 0)),
                pl.BlockSpec((S, H), lambda j, eid, nu: (0, 0)),
                pl.BlockSpec((1, H, I), lambda j, eid, nu: (eid[j], 0, 0)),
                pl.BlockSpec((1, 1, I), lambda j, eid, nu: (eid[j], 0, 0)),
                pl.BlockSpec((1, I, H), lambda j, eid, nu: (eid[j], 0, 0)),
                pl.BlockSpec((1, 1, H), lambda j, eid, nu: (eid[j], 0, 0)),
            ],
            out_specs=pl.BlockSpec((TT, H), lambda j, eid, nu: (j, 0)),
        ),
        out_shape=jax.ShapeDtypeStruct((PAD, H), jnp.float32),
    )(tile_eid, num_used, slot_row, xbf, w1, b1.reshape(E, 1, I),
      w2, b2.reshape(E, 1, H))

    out = _sc_unperm(buf, slot.reshape(S))

    return out.reshape(1, S, H)


# speedup vs baseline: 1.0552x; 1.0552x over previous
import functools

import jax
import jax.numpy as jnp
from jax import lax
from jax.experimental import pallas as pl
from jax.experimental.pallas import tpu as pltpu
from jax.experimental.pallas import tpu_sc as plsc

S = 2048
H = 768
I = 3072
E = 8
TT = 128
NT = 23
NIT = 6
IT = I // NIT
PAD = NT * TT


def _routing_body(x_ref, rw_ref, rb_ref, slot_ref, meta_ref):
    x = x_ref[...]
    logits = jnp.dot(x, rw_ref[...], preferred_element_type=jnp.float32) + rb_ref[...]
    col = jax.lax.broadcasted_iota(jnp.int32, (S, E), 1)
    m1 = jnp.max(logits, axis=1, keepdims=True)
    i1 = jnp.min(jnp.where(logits == m1, col, E), axis=1, keepdims=True)
    l2 = jnp.where(col == i1, -jnp.inf, logits)
    m2 = jnp.max(l2, axis=1, keepdims=True)
    i2 = jnp.min(jnp.where(l2 == m2, col, E), axis=1, keepdims=True)
    estar = jnp.maximum(i1, i2)

    oh = (estar == col)
    oh_f = oh.astype(jnp.float32)
    oh_b = oh.astype(jnp.bfloat16)
    r_i = jax.lax.broadcasted_iota(jnp.int32, (S, S), 0)
    c_i = jax.lax.broadcasted_iota(jnp.int32, (S, S), 1)
    lt = (c_i < r_i).astype(jnp.bfloat16)
    rank = jnp.dot(lt, oh_b, preferred_element_type=jnp.float32)

    counts = jnp.sum(oh_f, axis=0, keepdims=True).astype(jnp.int32)
    nt = (counts + (TT - 1)) >> 7
    nt_f = nt.astype(jnp.float32)
    a_i = jax.lax.broadcasted_iota(jnp.int32, (E, E), 0)
    b_i = jax.lax.broadcasted_iota(jnp.int32, (E, E), 1)
    m8 = (a_i < b_i).astype(jnp.float32)
    excl_f = jnp.dot(nt_f, m8, preferred_element_type=jnp.float32)
    incl_i = (excl_f + nt_f).astype(jnp.int32)
    start_rows = excl_f * float(TT)

    slot_f = jnp.sum(oh_f * (rank + start_rows), axis=1, keepdims=True)
    slot_ref[...] = slot_f.astype(jnp.int32)

    num_used = jnp.sum(nt, axis=1, keepdims=True)
    col8 = jax.lax.broadcasted_iota(jnp.int32, (1, E), 1)
    maxe = jnp.max(jnp.where(counts > 0, col8, 0), axis=1, keepdims=True)
    jv = jax.lax.broadcasted_iota(jnp.int32, (TT, E), 0)
    raw = jnp.sum((jv >= incl_i).astype(jnp.int32), axis=1, keepdims=True)
    eid = jnp.where(raw == E, maxe, raw)
    jv1 = jax.lax.broadcasted_iota(jnp.int32, (TT, 1), 0)
    meta_ref[...] = jnp.where(jv1 == NT, num_used, eid)


def _ffn_body(eid_ref, nu_ref, xs_ref, w1_ref, b1_ref, w2_ref,
              b2_ref, out_ref, w1b_ref, w2b_ref):
    j = pl.program_id(0)
    prev = eid_ref[jnp.maximum(j - 1, 0)]
    changed = jnp.logical_or(j == 0, eid_ref[j] != prev)

    @pl.when(j < nu_ref[0])
    def _():
        @pl.when(changed)
        def _():
            w1b_ref[...] = w1_ref[0].astype(jnp.bfloat16)
            w2b_ref[...] = w2_ref[0].astype(jnp.bfloat16)

        xg = xs_ref[...].astype(jnp.bfloat16)
        h = jnp.dot(xg, w1b_ref[...], preferred_element_type=jnp.float32)
        h = h + b1_ref[0]
        h = 0.5 * h * (1.0 + jax.lax.erf(h * 0.7071067811865476))
        out_ref[...] = jnp.dot(h.astype(jnp.bfloat16), w2b_ref[...],
                               preferred_element_type=jnp.float32) + b2_ref[0]


NWORK = 32
RPW = S // NWORK


def _sc_scatter_body(x_hbm, slot_hbm, xs_hbm, idx_v, rows_v, sem):
    wid = lax.axis_index("s") * 2 + lax.axis_index("c")
    base = wid * RPW
    pltpu.sync_copy(slot_hbm.at[pl.ds(base, RPW)], idx_v)
    pltpu.sync_copy(x_hbm.at[pl.ds(base, RPW)], rows_v)
    pltpu.async_copy(rows_v, xs_hbm.at[idx_v], sem).wait()


_sc_scatter = functools.partial(
    pl.kernel,
    mesh=plsc.VectorSubcoreMesh(core_axis_name="c", subcore_axis_name="s"),
    out_type=jax.ShapeDtypeStruct((PAD, H), jnp.float32),
    scratch_types=[
        pltpu.VMEM((RPW,), jnp.int32),
        pltpu.VMEM((RPW, H), jnp.float32),
        pltpu.SemaphoreType.DMA,
    ],
)(_sc_scatter_body)


def _sc_unperm_body(buf_hbm, slot_hbm, out_hbm, idx_v, rows_v, sem):
    wid = lax.axis_index("s") * 2 + lax.axis_index("c")
    base = wid * RPW
    pltpu.sync_copy(slot_hbm.at[pl.ds(base, RPW)], idx_v)
    pltpu.async_copy(buf_hbm.at[idx_v], rows_v, sem).wait()
    pltpu.sync_copy(rows_v, out_hbm.at[pl.ds(base, RPW)])


_sc_unperm = functools.partial(
    pl.kernel,
    mesh=plsc.VectorSubcoreMesh(core_axis_name="c", subcore_axis_name="s"),
    out_type=jax.ShapeDtypeStruct((S, H), jnp.float32),
    scratch_types=[
        pltpu.VMEM((RPW,), jnp.int32),
        pltpu.VMEM((RPW, H), jnp.float32),
        pltpu.SemaphoreType.DMA,
    ],
)(_sc_unperm_body)


def kernel(hidden_states, router_w, router_b, w1, b1, w2, b2):
    x = hidden_states.reshape(S, H)
    rb = router_b.reshape(1, E)

    slot, meta = pl.pallas_call(
        _routing_body,
        out_shape=[
            jax.ShapeDtypeStruct((S, 1), jnp.int32),
            jax.ShapeDtypeStruct((TT, 1), jnp.int32),
        ],
    )(x, router_w, rb)

    tile_eid = meta[:NT, 0]
    num_used = meta[NT, 0].reshape(1)
    slot1d = slot.reshape(S)

    xs = _sc_scatter(x, slot1d)

    buf = pl.pallas_call(
        _ffn_body,
        grid_spec=pltpu.PrefetchScalarGridSpec(
            num_scalar_prefetch=2,
            grid=(NT,),
            in_specs=[
                pl.BlockSpec((TT, H), lambda j, eid, nu: (j, 0)),
                pl.BlockSpec((1, H, I), lambda j, eid, nu: (eid[j], 0, 0)),
                pl.BlockSpec((1, 1, I), lambda j, eid, nu: (eid[j], 0, 0)),
                pl.BlockSpec((1, I, H), lambda j, eid, nu: (eid[j], 0, 0)),
                pl.BlockSpec((1, 1, H), lambda j, eid, nu: (eid[j], 0, 0)),
            ],
            out_specs=pl.BlockSpec((TT, H), lambda j, eid, nu: (j, 0)),
            scratch_shapes=[
                pltpu.VMEM((H, I), jnp.bfloat16),
                pltpu.VMEM((I, H), jnp.bfloat16),
            ],
        ),
        out_shape=jax.ShapeDtypeStruct((PAD, H), jnp.float32),
    )(tile_eid, num_used, xs, w1, b1.reshape(E, 1, I),
      w2, b2.reshape(E, 1, H))

    out = _sc_unperm(buf, slot1d)

    return out.reshape(1, S, H)


# speedup vs baseline: 1.1334x; 1.0741x over previous
import functools

import jax
import jax.numpy as jnp
from jax import lax
from jax.experimental import pallas as pl
from jax.experimental.pallas import tpu as pltpu
from jax.experimental.pallas import tpu_sc as plsc

S = 2048
H = 768
I = 3072
E = 8
TT = 256
NT = 15
NIT = 6
IT = I // NIT
PAD = NT * TT


def _routing_body(x_ref, rw_ref, rb_ref, slot_ref, meta_ref):
    x = x_ref[...]
    logits = jnp.dot(x, rw_ref[...], preferred_element_type=jnp.float32) + rb_ref[...]
    col = jax.lax.broadcasted_iota(jnp.int32, (S, E), 1)
    m1 = jnp.max(logits, axis=1, keepdims=True)
    i1 = jnp.min(jnp.where(logits == m1, col, E), axis=1, keepdims=True)
    l2 = jnp.where(col == i1, -jnp.inf, logits)
    m2 = jnp.max(l2, axis=1, keepdims=True)
    i2 = jnp.min(jnp.where(l2 == m2, col, E), axis=1, keepdims=True)
    estar = jnp.maximum(i1, i2)

    oh = (estar == col)
    oh_f = oh.astype(jnp.float32)
    oh_b = oh.astype(jnp.bfloat16)
    r_i = jax.lax.broadcasted_iota(jnp.int32, (S, S), 0)
    c_i = jax.lax.broadcasted_iota(jnp.int32, (S, S), 1)
    lt = (c_i < r_i).astype(jnp.bfloat16)
    rank = jnp.dot(lt, oh_b, preferred_element_type=jnp.float32)

    counts = jnp.sum(oh_f, axis=0, keepdims=True).astype(jnp.int32)
    nt = (counts + (TT - 1)) >> 8
    nt_f = nt.astype(jnp.float32)
    a_i = jax.lax.broadcasted_iota(jnp.int32, (E, E), 0)
    b_i = jax.lax.broadcasted_iota(jnp.int32, (E, E), 1)
    m8 = (a_i < b_i).astype(jnp.float32)
    excl_f = jnp.dot(nt_f, m8, preferred_element_type=jnp.float32)
    incl_i = (excl_f + nt_f).astype(jnp.int32)
    start_rows = excl_f * float(TT)

    slot_f = jnp.sum(oh_f * (rank + start_rows), axis=1, keepdims=True)
    slot_ref[...] = slot_f.astype(jnp.int32)

    num_used = jnp.sum(nt, axis=1, keepdims=True)
    col8 = jax.lax.broadcasted_iota(jnp.int32, (1, E), 1)
    maxe = jnp.max(jnp.where(counts > 0, col8, 0), axis=1, keepdims=True)
    jv = jax.lax.broadcasted_iota(jnp.int32, (TT, E), 0)
    raw = jnp.sum((jv >= incl_i).astype(jnp.int32), axis=1, keepdims=True)
    eid = jnp.where(raw == E, maxe, raw)
    jv1 = jax.lax.broadcasted_iota(jnp.int32, (TT, 1), 0)
    meta_ref[...] = jnp.where(jv1 == NT, num_used, eid)


def _ffn_body(eid_ref, nu_ref, xs_ref, w1_ref, b1_ref, w2_ref,
              b2_ref, out_ref, w1b_ref, w2b_ref):
    j = pl.program_id(0)
    prev = eid_ref[jnp.maximum(j - 1, 0)]
    changed = jnp.logical_or(j == 0, eid_ref[j] != prev)

    @pl.when(j < nu_ref[0])
    def _():
        @pl.when(changed)
        def _():
            w1b_ref[...] = w1_ref[0].astype(jnp.bfloat16)
            w2b_ref[...] = w2_ref[0].astype(jnp.bfloat16)

        xg = xs_ref[...].astype(jnp.bfloat16)
        h = jnp.dot(xg, w1b_ref[...], preferred_element_type=jnp.float32)
        h = h + b1_ref[0]
        h = 0.5 * h * (1.0 + jax.lax.erf(h * 0.7071067811865476))
        out_ref[...] = jnp.dot(h.astype(jnp.bfloat16), w2b_ref[...],
                               preferred_element_type=jnp.float32) + b2_ref[0]


NWORK = 32
RPW = S // NWORK


def _sc_scatter_body(x_hbm, slot_hbm, xs_hbm, idx_v, rows_v, sem):
    wid = lax.axis_index("s") * 2 + lax.axis_index("c")
    base = wid * RPW
    pltpu.sync_copy(slot_hbm.at[pl.ds(base, RPW)], idx_v)
    pltpu.sync_copy(x_hbm.at[pl.ds(base, RPW)], rows_v)
    pltpu.async_copy(rows_v, xs_hbm.at[idx_v], sem).wait()


_sc_scatter = functools.partial(
    pl.kernel,
    mesh=plsc.VectorSubcoreMesh(core_axis_name="c", subcore_axis_name="s"),
    out_type=jax.ShapeDtypeStruct((PAD, H), jnp.float32),
    scratch_types=[
        pltpu.VMEM((RPW,), jnp.int32),
        pltpu.VMEM((RPW, H), jnp.float32),
        pltpu.SemaphoreType.DMA,
    ],
)(_sc_scatter_body)


def _sc_unperm_body(buf_hbm, slot_hbm, out_hbm, idx_v, rows_v, sem):
    wid = lax.axis_index("s") * 2 + lax.axis_index("c")
    base = wid * RPW
    pltpu.sync_copy(slot_hbm.at[pl.ds(base, RPW)], idx_v)
    pltpu.async_copy(buf_hbm.at[idx_v], rows_v, sem).wait()
    pltpu.sync_copy(rows_v, out_hbm.at[pl.ds(base, RPW)])


_sc_unperm = functools.partial(
    pl.kernel,
    mesh=plsc.VectorSubcoreMesh(core_axis_name="c", subcore_axis_name="s"),
    out_type=jax.ShapeDtypeStruct((S, H), jnp.float32),
    scratch_types=[
        pltpu.VMEM((RPW,), jnp.int32),
        pltpu.VMEM((RPW, H), jnp.float32),
        pltpu.SemaphoreType.DMA,
    ],
)(_sc_unperm_body)


def kernel(hidden_states, router_w, router_b, w1, b1, w2, b2):
    x = hidden_states.reshape(S, H)
    rb = router_b.reshape(1, E)

    slot, meta = pl.pallas_call(
        _routing_body,
        out_shape=[
            jax.ShapeDtypeStruct((S, 1), jnp.int32),
            jax.ShapeDtypeStruct((TT, 1), jnp.int32),
        ],
    )(x, router_w, rb)

    tile_eid = meta[:NT, 0]
    num_used = meta[NT, 0].reshape(1)
    slot1d = slot.reshape(S)

    xs = _sc_scatter(x, slot1d)

    buf = pl.pallas_call(
        _ffn_body,
        grid_spec=pltpu.PrefetchScalarGridSpec(
            num_scalar_prefetch=2,
            grid=(NT,),
            in_specs=[
                pl.BlockSpec((TT, H), lambda j, eid, nu: (j, 0)),
                pl.BlockSpec((1, H, I), lambda j, eid, nu: (eid[j], 0, 0)),
                pl.BlockSpec((1, 1, I), lambda j, eid, nu: (eid[j], 0, 0)),
                pl.BlockSpec((1, I, H), lambda j, eid, nu: (eid[j], 0, 0)),
                pl.BlockSpec((1, 1, H), lambda j, eid, nu: (eid[j], 0, 0)),
            ],
            out_specs=pl.BlockSpec((TT, H), lambda j, eid, nu: (j, 0)),
            scratch_shapes=[
                pltpu.VMEM((H, I), jnp.bfloat16),
                pltpu.VMEM((I, H), jnp.bfloat16),
            ],
        ),
        out_shape=jax.ShapeDtypeStruct((PAD, H), jnp.float32),
    )(tile_eid, num_used, xs, w1, b1.reshape(E, 1, I),
      w2, b2.reshape(E, 1, H))

    out = _sc_unperm(buf, slot1d)

    return out.reshape(1, S, H)


# speedup vs baseline: 1.1716x; 1.0337x over previous
import functools

import jax
import jax.numpy as jnp
from jax import lax
from jax.experimental import pallas as pl
from jax.experimental.pallas import tpu as pltpu
from jax.experimental.pallas import tpu_sc as plsc

S = 2048
H = 768
I = 3072
E = 8
TT = 256
NT = 15
NIT = 6
IT = I // NIT
PAD = NT * TT


def _routing_body(x_ref, rw_ref, rb_ref, slot_ref, meta_ref):
    x = x_ref[...]
    logits = jnp.dot(x, rw_ref[...], preferred_element_type=jnp.float32) + rb_ref[...]
    col = jax.lax.broadcasted_iota(jnp.int32, (S, E), 1)
    m1 = jnp.max(logits, axis=1, keepdims=True)
    i1 = jnp.min(jnp.where(logits == m1, col, E), axis=1, keepdims=True)
    l2 = jnp.where(col == i1, -jnp.inf, logits)
    m2 = jnp.max(l2, axis=1, keepdims=True)
    i2 = jnp.min(jnp.where(l2 == m2, col, E), axis=1, keepdims=True)
    estar = jnp.maximum(i1, i2)

    oh = (estar == col)
    oh_f = oh.astype(jnp.float32)
    zrow = jnp.zeros((1, E), jnp.float32)
    rank = jnp.concatenate([zrow, oh_f[:-1]], axis=0)
    k = 1
    while k < S:
        rank = rank + jnp.concatenate(
            [jnp.zeros((k, E), jnp.float32), rank[:-k]], axis=0)
        k *= 2

    counts = jnp.sum(oh_f, axis=0, keepdims=True).astype(jnp.int32)
    nt = (counts + (TT - 1)) >> 8
    nt_f = nt.astype(jnp.float32)
    a_i = jax.lax.broadcasted_iota(jnp.int32, (E, E), 0)
    b_i = jax.lax.broadcasted_iota(jnp.int32, (E, E), 1)
    m8 = (a_i < b_i).astype(jnp.float32)
    excl_f = jnp.dot(nt_f, m8, preferred_element_type=jnp.float32)
    incl_i = (excl_f + nt_f).astype(jnp.int32)
    start_rows = excl_f * float(TT)

    slot_f = jnp.sum(oh_f * (rank + start_rows), axis=1, keepdims=True)
    slot_ref[...] = slot_f.astype(jnp.int32)

    num_used = jnp.sum(nt, axis=1, keepdims=True)
    col8 = jax.lax.broadcasted_iota(jnp.int32, (1, E), 1)
    maxe = jnp.max(jnp.where(counts > 0, col8, 0), axis=1, keepdims=True)
    jv = jax.lax.broadcasted_iota(jnp.int32, (TT, E), 0)
    raw = jnp.sum((jv >= incl_i).astype(jnp.int32), axis=1, keepdims=True)
    eid = jnp.where(raw == E, maxe, raw)
    jv1 = jax.lax.broadcasted_iota(jnp.int32, (TT, 1), 0)
    meta_ref[...] = jnp.where(jv1 == NT, num_used, eid)


def _ffn_body(eid_ref, nu_ref, xs_ref, w1_ref, b1_ref, w2_ref,
              b2_ref, out_ref, w1b_ref, w2b_ref):
    j = pl.program_id(0)
    prev = eid_ref[jnp.maximum(j - 1, 0)]
    changed = jnp.logical_or(j == 0, eid_ref[j] != prev)

    @pl.when(j < nu_ref[0])
    def _():
        @pl.when(changed)
        def _():
            w1b_ref[...] = w1_ref[0].astype(jnp.bfloat16)
            w2b_ref[...] = w2_ref[0].astype(jnp.bfloat16)

        xg = xs_ref[...].astype(jnp.bfloat16)
        h = jnp.dot(xg, w1b_ref[...], preferred_element_type=jnp.float32)
        h = h + b1_ref[0]
        h = 0.5 * h * (1.0 + jax.lax.erf(h * 0.7071067811865476))
        out_ref[...] = jnp.dot(h.astype(jnp.bfloat16), w2b_ref[...],
                               preferred_element_type=jnp.float32) + b2_ref[0]


NWORK = 32
RPW = S // NWORK


def _sc_scatter_body(x_hbm, slot_hbm, xs_hbm, idx_v, rows_v, sem):
    wid = lax.axis_index("s") * 2 + lax.axis_index("c")
    base = wid * RPW
    pltpu.sync_copy(slot_hbm.at[pl.ds(base, RPW)], idx_v)
    pltpu.sync_copy(x_hbm.at[pl.ds(base, RPW)], rows_v)
    pltpu.async_copy(rows_v, xs_hbm.at[idx_v], sem).wait()


_sc_scatter = functools.partial(
    pl.kernel,
    mesh=plsc.VectorSubcoreMesh(core_axis_name="c", subcore_axis_name="s"),
    out_type=jax.ShapeDtypeStruct((PAD, H), jnp.float32),
    scratch_types=[
        pltpu.VMEM((RPW,), jnp.int32),
        pltpu.VMEM((RPW, H), jnp.float32),
        pltpu.SemaphoreType.DMA,
    ],
)(_sc_scatter_body)


def _sc_unperm_body(buf_hbm, slot_hbm, out_hbm, idx_v, rows_v, sem):
    wid = lax.axis_index("s") * 2 + lax.axis_index("c")
    base = wid * RPW
    pltpu.sync_copy(slot_hbm.at[pl.ds(base, RPW)], idx_v)
    pltpu.async_copy(buf_hbm.at[idx_v], rows_v, sem).wait()
    pltpu.sync_copy(rows_v, out_hbm.at[pl.ds(base, RPW)])


_sc_unperm = functools.partial(
    pl.kernel,
    mesh=plsc.VectorSubcoreMesh(core_axis_name="c", subcore_axis_name="s"),
    out_type=jax.ShapeDtypeStruct((S, H), jnp.float32),
    scratch_types=[
        pltpu.VMEM((RPW,), jnp.int32),
        pltpu.VMEM((RPW, H), jnp.float32),
        pltpu.SemaphoreType.DMA,
    ],
)(_sc_unperm_body)


def kernel(hidden_states, router_w, router_b, w1, b1, w2, b2):
    x = hidden_states.reshape(S, H)
    rb = router_b.reshape(1, E)

    slot, meta = pl.pallas_call(
        _routing_body,
        out_shape=[
            jax.ShapeDtypeStruct((S, 1), jnp.int32),
            jax.ShapeDtypeStruct((TT, 1), jnp.int32),
        ],
    )(x, router_w, rb)

    tile_eid = meta[:NT, 0]
    num_used = meta[NT, 0].reshape(1)
    slot1d = slot.reshape(S)

    xs = _sc_scatter(x, slot1d)

    buf = pl.pallas_call(
        _ffn_body,
        grid_spec=pltpu.PrefetchScalarGridSpec(
            num_scalar_prefetch=2,
            grid=(NT,),
            in_specs=[
                pl.BlockSpec((TT, H), lambda j, eid, nu: (j, 0)),
                pl.BlockSpec((1, H, I), lambda j, eid, nu: (eid[j], 0, 0)),
                pl.BlockSpec((1, 1, I), lambda j, eid, nu: (eid[j], 0, 0)),
                pl.BlockSpec((1, I, H), lambda j, eid, nu: (eid[j], 0, 0)),
                pl.BlockSpec((1, 1, H), lambda j, eid, nu: (eid[j], 0, 0)),
            ],
            out_specs=pl.BlockSpec((TT, H), lambda j, eid, nu: (j, 0)),
            scratch_shapes=[
                pltpu.VMEM((H, I), jnp.bfloat16),
                pltpu.VMEM((I, H), jnp.bfloat16),
            ],
        ),
        out_shape=jax.ShapeDtypeStruct((PAD, H), jnp.float32),
    )(tile_eid, num_used, xs, w1, b1.reshape(E, 1, I),
      w2, b2.reshape(E, 1, H))

    out = _sc_unperm(buf, slot1d)

    return out.reshape(1, S, H)
